# baseline (speedup 1.0000x reference)
import functools

import numpy as np

import jax
import jax.numpy as jnp
from jax import lax
from jax.experimental import pallas as pl
from jax.experimental.pallas import tpu as pltpu
from jax.experimental.pallas import tpu_sc as plsc

_NC = 2
_NS = 16
_NW = _NC * _NS
_CH = 128


def _make_seg_sum(n_pad, d, n0, n1, with_counts):
    rpt = n_pad // _NS
    ncp = rpt // _CH
    mesh = plsc.VectorSubcoreMesh(core_axis_name="c", subcore_axis_name="s")

    outs = jax.ShapeDtypeStruct((_NC, n_pad, d), jnp.float32)
    scratch = [
        pltpu.VMEM((n0, _CH), jnp.int32),
        pltpu.VMEM((n0, _CH), jnp.int32),
        pltpu.VMEM((_CH, d // 2), jnp.int32),
        pltpu.VMEM((_CH, d // 2), jnp.int32),
        pltpu.VMEM((_CH, d), jnp.float32),
        pltpu.VMEM((_CH, d), jnp.float32),
        pltpu.VMEM_SHARED((n_pad, d), jnp.float32),
        pltpu.SemaphoreType.DMA,
        pltpu.SemaphoreType.DMA,
        pltpu.SemaphoreType.DMA,
        pltpu.SemaphoreType.DMA,
    ]
    if with_counts:
        outs = [outs, jax.ShapeDtypeStruct((_NC, n_pad, 16), jnp.float32)]
        scratch += [
            pltpu.VMEM((_CH, 16), jnp.float32),
            pltpu.VMEM_SHARED((n_pad, 16), jnp.float32),
            pltpu.SemaphoreType.DMA,
        ]

    def load_idx(src_hbm, dst_hbm, c, s, src_v, dst_v):
        @pl.when(c == 0)
        def _():
            pltpu.sync_copy(src_hbm.at[s], src_v)
            pltpu.sync_copy(dst_hbm.at[s], dst_v)

        @pl.when(c == 1)
        def _():
            pltpu.sync_copy(src_hbm.at[_NS + s, pl.ds(0, n1)],
                            src_v.at[pl.ds(0, n1)])
            pltpu.sync_copy(dst_hbm.at[_NS + s, pl.ds(0, n1)],
                            dst_v.at[pl.ds(0, n1)])

    def pipeline(rows_hbm, src_v, dst_v, bfA, bfB, bufA, bufB, acc_sh,
                 gsA, gsB, ssA, ssB, nch, per_chunk):
        pltpu.async_copy(rows_hbm.at[src_v.at[0]], bfA, gsA)
        pltpu.async_copy(rows_hbm.at[src_v.at[1]], bfB, gsB)

        def convert(bf, buf):
            @plsc.parallel_loop(0, _CH, unroll=4)
            def crow(i):
                for g in range(d // 32):
                    vi = bf[i, pl.ds(16 * g, 16)]
                    lo = lax.bitcast_convert_type(
                        lax.shift_left(vi, 16), jnp.float32)
                    hi = lax.bitcast_convert_type(
                        lax.bitwise_and(vi, jnp.int32(-65536)), jnp.float32)
                    buf[i, pl.ds(16 * g, 16)] = lo
                    buf[i, pl.ds(d // 2 + 16 * g, 16)] = hi

        def half(j, bf, buf, gs, ss):
            pltpu.make_async_copy(rows_hbm.at[src_v.at[j]], bf, gs).wait()
            convert(bf, buf)

            @pl.when(j + 2 < nch)
            def _():
                pltpu.async_copy(rows_hbm.at[src_v.at[j + 2]], bf, gs)

            dsc = pltpu.async_copy(buf, acc_sh.at[dst_v.at[j]], ss, add=True)
            per_chunk(j)
            dsc.wait()

        def step(t, carry):
            half(2 * t, bfA, bufA, gsA, ssA)
            half(2 * t + 1, bfB, bufB, gsB, ssB)
            return carry
        lax.fori_loop(0, nch // 2, step, 0)

    def body_counts(rows_hbm, src_hbm, dst_hbm, out_hbm, cnt_hbm,
                    src_v, dst_v, bfA, bfB, bufA, bufB, acc_sh,
                    gsA, gsB, ssA, ssB, w16_v, cnt_sh, csem):
        zero16 = jnp.zeros((16,), jnp.float32)
        one16 = jnp.ones((16,), jnp.float32)
        c = lax.axis_index("c")
        s = lax.axis_index("s")
        nch = jnp.where(c == 0, n0, n1)
        load_idx(src_hbm, dst_hbm, c, s, src_v, dst_v)

        def zrow(i, carry):
            for cc in range(d // 16):
                bufA[i, pl.ds(cc * 16, 16)] = zero16
            w16_v[i, :] = zero16
            return carry
        lax.fori_loop(0, _CH, zrow, 0)

        r0 = s * rpt
        for i in range(ncp):
            sl = pl.ds(r0 + i * _CH, _CH)
            pltpu.sync_copy(bufA, acc_sh.at[sl])
            pltpu.sync_copy(w16_v, cnt_sh.at[sl])

        def orow(i, carry):
            w16_v[i, :] = one16
            return carry
        lax.fori_loop(0, _CH, orow, 0)
        plsc.subcore_barrier()

        def per_chunk(j):
            pltpu.async_copy(w16_v, cnt_sh.at[dst_v.at[j]], csem, add=True)

        pipeline(rows_hbm, src_v, dst_v, bfA, bfB, bufA, bufB, acc_sh,
                 gsA, gsB, ssA, ssB, nch, per_chunk)

        def drain(j, carry):
            pltpu.make_async_copy(w16_v, cnt_sh.at[dst_v.at[0]], csem).wait()
            return carry
        lax.fori_loop(0, nch, drain, 0)
        plsc.subcore_barrier()

        for i in range(ncp):
            sl = pl.ds(r0 + i * _CH, _CH)
            pltpu.sync_copy(acc_sh.at[sl], bufA)
            pltpu.sync_copy(bufA, out_hbm.at[c, sl])
            pltpu.sync_copy(cnt_sh.at[sl], w16_v)
            pltpu.sync_copy(w16_v, cnt_hbm.at[c, sl])

    def body_plain(rows_hbm, src_hbm, dst_hbm, out_hbm,
                   src_v, dst_v, bfA, bfB, bufA, bufB, acc_sh,
                   gsA, gsB, ssA, ssB):
        zero16 = jnp.zeros((16,), jnp.float32)
        c = lax.axis_index("c")
        s = lax.axis_index("s")
        nch = jnp.where(c == 0, n0, n1)
        load_idx(src_hbm, dst_hbm, c, s, src_v, dst_v)

        def zrow(i, carry):
            for cc in range(d // 16):
                bufA[i, pl.ds(cc * 16, 16)] = zero16
            return carry
        lax.fori_loop(0, _CH, zrow, 0)

        r0 = s * rpt
        for i in range(ncp):
            pltpu.sync_copy(bufA, acc_sh.at[pl.ds(r0 + i * _CH, _CH)])
        plsc.subcore_barrier()

        pipeline(rows_hbm, src_v, dst_v, bfA, bfB, bufA, bufB, acc_sh,
                 gsA, gsB, ssA, ssB, nch, lambda j: None)
        plsc.subcore_barrier()

        for i in range(ncp):
            sl = pl.ds(r0 + i * _CH, _CH)
            pltpu.sync_copy(acc_sh.at[sl], bufA)
            pltpu.sync_copy(bufA, out_hbm.at[c, sl])

    body = body_counts if with_counts else body_plain
    return pl.kernel(
        body, mesh=mesh, out_type=outs, scratch_types=scratch,
        compiler_params=pltpu.CompilerParams(use_tc_tiling_on_sc=False))


def _pack_bf16_pairs(xf):
    k = xf.shape[1] // 2
    one = jnp.uint32(1)
    half = jnp.uint32(0x7FFF)
    sixteen = jnp.uint32(16)
    ul = lax.bitcast_convert_type(xf[:, :k], jnp.uint32)
    ur = lax.bitcast_convert_type(xf[:, k:], jnp.uint32)
    tl = lax.shift_right_logical(
        ul + half + (lax.shift_right_logical(ul, sixteen) & one), sixteen)
    tr = lax.shift_right_logical(
        ur + half + (lax.shift_right_logical(ur, sixteen) & one), sixteen)
    return lax.bitcast_convert_type(
        tl | lax.shift_left(tr, sixteen), jnp.int32)


def _k1(x_ref, w_ref, o_ref):
    o_ref[...] = _pack_bf16_pairs(
        jnp.dot(x_ref[...], w_ref[...], preferred_element_type=jnp.float32))


def _k3(p_ref, c_ref, x_ref, w1root_ref, b1_ref, w2rel_ref, h_ref, hr_ref):
    n = x_ref.shape[0]
    agg = p_ref[0, :n, :] + p_ref[1, :n, :]
    cnt = c_ref[0, :n, 0:1] + c_ref[1, :n, 0:1]
    inv = 1.0 / jnp.maximum(cnt, 1.0)
    root = jnp.dot(x_ref[...], w1root_ref[...],
                   preferred_element_type=jnp.float32)
    h = jnp.maximum(agg * inv + b1_ref[...][None, :] + root, 0.0)
    h_ref[...] = h
    hr_ref[...] = _pack_bf16_pairs(
        jnp.dot(h, w2rel_ref[...], preferred_element_type=jnp.float32))


def _k5(p_ref, c_ref, h_ref, w2root_ref, b2_ref, wl_ref, bl_ref, o_ref):
    n = h_ref.shape[0]
    agg = p_ref[0, :n, :] + p_ref[1, :n, :]
    cnt = c_ref[0, :n, 0:1] + c_ref[1, :n, 0:1]
    inv = 1.0 / jnp.maximum(cnt, 1.0)
    root = jnp.dot(h_ref[...], w2root_ref[...],
                   preferred_element_type=jnp.float32)
    h2 = jnp.maximum(agg * inv + b2_ref[...][None, :] + root, 0.0)
    o_ref[...] = jnp.dot(h2, wl_ref[...],
                         preferred_element_type=jnp.float32) + bl_ref[0]


def kernel(x, edge_index, W1_rel, b1, W1_root, W2_rel, b2, W2_root, Wl, bl):
    n, d_in = x.shape
    h1 = W1_rel.shape[1]
    h2 = W2_rel.shape[1]
    e = edge_index.shape[1]

    t_ch = -(-e // _CH)
    n0 = max(2, (int(t_ch * 0.655) // (_NS * 2)) * 2)
    n1 = max(2, (-(-(t_ch - _NS * n0) // (_NS * 2))) * 2)
    e0 = _NS * n0 * _CH
    e_pad = e0 + _NS * n1 * _CH
    n_pad = -(-(n + 1) // (_NS * _CH)) * (_NS * _CH)

    def _pack(a, fill):
        flat = jnp.concatenate(
            [a, jnp.full((e_pad - e,), fill, jnp.int32)])
        p0 = flat[:e0].reshape(_NS, n0, _CH)
        p1 = flat[e0:].reshape(_NS, n1, _CH)
        p1 = jnp.pad(p1, ((0, 0), (0, n0 - n1), (0, 0)))
        return jnp.concatenate([p0, p1], axis=0)

    src = _pack(edge_index[0], 0)
    dst = _pack(edge_index[1], n)

    xr = pl.pallas_call(
        _k1, out_shape=jax.ShapeDtypeStruct((n, h1 // 2), jnp.int32))(
            x, W1_rel)

    p1, cnt = _make_seg_sum(n_pad, h1, n0, n1, True)(xr, src, dst)

    h, hr = pl.pallas_call(
        _k3,
        out_shape=[jax.ShapeDtypeStruct((n, h1), jnp.float32),
                   jax.ShapeDtypeStruct((n, h2 // 2), jnp.int32)],
    )(p1, cnt, x, W1_root, b1, W2_rel)

    p2 = _make_seg_sum(n_pad, h2, n0, n1, False)(hr, src, dst)

    out = pl.pallas_call(
        _k5,
        out_shape=jax.ShapeDtypeStruct((n, 1), jnp.float32),
    )(p2, cnt, h, W2_root, b2, Wl, bl)
    return out


# speedup vs baseline: 1.0212x; 1.0212x over previous
import functools

import numpy as np

import jax
import jax.numpy as jnp
from jax import lax
from jax.experimental import pallas as pl
from jax.experimental.pallas import tpu as pltpu
from jax.experimental.pallas import tpu_sc as plsc

_NC = 2
_NS = 16
_NW = _NC * _NS
_CH = 128


def _make_seg_sum(n_pad, d, n0, n1, with_counts):
    rpt = n_pad // _NS
    ncp = rpt // _CH
    dd = d + 16 if with_counts else d
    mesh = plsc.VectorSubcoreMesh(core_axis_name="c", subcore_axis_name="s")

    outs = jax.ShapeDtypeStruct((_NC, n_pad, dd), jnp.float32)
    scratch = [
        pltpu.VMEM((n0, _CH), jnp.int32),
        pltpu.VMEM((n0, _CH), jnp.int32),
        pltpu.VMEM((_CH, d // 2), jnp.int32),
        pltpu.VMEM((_CH, d // 2), jnp.int32),
        pltpu.VMEM((_CH, dd), jnp.float32),
        pltpu.VMEM((_CH, dd), jnp.float32),
        pltpu.VMEM_SHARED((n_pad, dd), jnp.float32),
        pltpu.SemaphoreType.DMA,
        pltpu.SemaphoreType.DMA,
        pltpu.SemaphoreType.DMA,
        pltpu.SemaphoreType.DMA,
    ]

    def load_idx(src_hbm, dst_hbm, c, s, src_v, dst_v):
        @pl.when(c == 0)
        def _():
            pltpu.sync_copy(src_hbm.at[s], src_v)
            pltpu.sync_copy(dst_hbm.at[s], dst_v)

        @pl.when(c == 1)
        def _():
            pltpu.sync_copy(src_hbm.at[_NS + s, pl.ds(0, n1)],
                            src_v.at[pl.ds(0, n1)])
            pltpu.sync_copy(dst_hbm.at[_NS + s, pl.ds(0, n1)],
                            dst_v.at[pl.ds(0, n1)])

    def pipeline(rows_hbm, src_v, dst_v, bfA, bfB, bufA, bufB, acc_sh,
                 gsA, gsB, ssA, ssB, nch, per_chunk):
        pltpu.async_copy(rows_hbm.at[src_v.at[0]], bfA, gsA)
        pltpu.async_copy(rows_hbm.at[src_v.at[1]], bfB, gsB)

        def convert(bf, buf):
            @plsc.parallel_loop(0, _CH, unroll=4)
            def crow(i):
                for g in range(d // 32):
                    vi = bf[i, pl.ds(16 * g, 16)]
                    lo = lax.bitcast_convert_type(
                        lax.shift_left(vi, 16), jnp.float32)
                    hi = lax.bitcast_convert_type(
                        lax.bitwise_and(vi, jnp.int32(-65536)), jnp.float32)
                    buf[i, pl.ds(16 * g, 16)] = lo
                    buf[i, pl.ds(d // 2 + 16 * g, 16)] = hi

        def half(j, bf, buf, gs, ss):
            pltpu.make_async_copy(rows_hbm.at[src_v.at[j]], bf, gs).wait()
            convert(bf, buf)

            @pl.when(j + 2 < nch)
            def _():
                pltpu.async_copy(rows_hbm.at[src_v.at[j + 2]], bf, gs)

            dsc = pltpu.async_copy(buf, acc_sh.at[dst_v.at[j]], ss, add=True)
            per_chunk(j)
            dsc.wait()

        def step(t, carry):
            half(2 * t, bfA, bufA, gsA, ssA)
            half(2 * t + 1, bfB, bufB, gsB, ssB)
            return carry
        lax.fori_loop(0, nch // 2, step, 0)

    def body(rows_hbm, src_hbm, dst_hbm, out_hbm,
             src_v, dst_v, bfA, bfB, bufA, bufB, acc_sh,
             gsA, gsB, ssA, ssB):
        zero16 = jnp.zeros((16,), jnp.float32)
        one16 = jnp.ones((16,), jnp.float32)
        c = lax.axis_index("c")
        s = lax.axis_index("s")
        nch = jnp.where(c == 0, n0, n1)
        load_idx(src_hbm, dst_hbm, c, s, src_v, dst_v)

        def zrow(i, carry):
            for cc in range(dd // 16):
                bufA[i, pl.ds(cc * 16, 16)] = zero16
            return carry
        lax.fori_loop(0, _CH, zrow, 0)

        r0 = s * rpt
        for i in range(ncp):
            pltpu.sync_copy(bufA, acc_sh.at[pl.ds(r0 + i * _CH, _CH)])

        if with_counts:
            def orow(i, carry):
                bufA[i, pl.ds(d, 16)] = one16
                bufB[i, pl.ds(d, 16)] = one16
                return carry
            lax.fori_loop(0, _CH, orow, 0)
        plsc.subcore_barrier()

        pipeline(rows_hbm, src_v, dst_v, bfA, bfB, bufA, bufB, acc_sh,
                 gsA, gsB, ssA, ssB, nch, lambda j: None)
        plsc.subcore_barrier()

        for i in range(ncp):
            sl = pl.ds(r0 + i * _CH, _CH)
            pltpu.sync_copy(acc_sh.at[sl], bufA)
            pltpu.sync_copy(bufA, out_hbm.at[c, sl])

    return pl.kernel(
        body, mesh=mesh, out_type=outs, scratch_types=scratch,
        compiler_params=pltpu.CompilerParams(use_tc_tiling_on_sc=False))


def _pack_bf16_pairs(xf):
    k = xf.shape[1] // 2
    one = jnp.uint32(1)
    half = jnp.uint32(0x7FFF)
    sixteen = jnp.uint32(16)
    ul = lax.bitcast_convert_type(xf[:, :k], jnp.uint32)
    ur = lax.bitcast_convert_type(xf[:, k:], jnp.uint32)
    tl = lax.shift_right_logical(
        ul + half + (lax.shift_right_logical(ul, sixteen) & one), sixteen)
    tr = lax.shift_right_logical(
        ur + half + (lax.shift_right_logical(ur, sixteen) & one), sixteen)
    return lax.bitcast_convert_type(
        tl | lax.shift_left(tr, sixteen), jnp.int32)


def _k1(x_ref, w_ref, o_ref):
    o_ref[...] = _pack_bf16_pairs(
        jnp.dot(x_ref[...], w_ref[...], preferred_element_type=jnp.float32))


def _k3(p_ref, x_ref, w1root_ref, b1_ref, w2rel_ref, h_ref, hr_ref):
    n = x_ref.shape[0]
    k = w1root_ref.shape[1]
    agg = p_ref[0, :n, :k] + p_ref[1, :n, :k]
    cnt = p_ref[0, :n, k:k + 1] + p_ref[1, :n, k:k + 1]
    inv = 1.0 / jnp.maximum(cnt, 1.0)
    root = jnp.dot(x_ref[...], w1root_ref[...],
                   preferred_element_type=jnp.float32)
    h = jnp.maximum(agg * inv + b1_ref[...][None, :] + root, 0.0)
    h_ref[...] = h
    hr_ref[...] = _pack_bf16_pairs(
        jnp.dot(h, w2rel_ref[...], preferred_element_type=jnp.float32))


def _k5(p_ref, c_ref, h_ref, w2root_ref, b2_ref, wl_ref, bl_ref, o_ref):
    n = h_ref.shape[0]
    k = h_ref.shape[1]
    agg = p_ref[0, :n, :] + p_ref[1, :n, :]
    cnt = c_ref[0, :n, k:k + 1] + c_ref[1, :n, k:k + 1]
    inv = 1.0 / jnp.maximum(cnt, 1.0)
    root = jnp.dot(h_ref[...], w2root_ref[...],
                   preferred_element_type=jnp.float32)
    h2 = jnp.maximum(agg * inv + b2_ref[...][None, :] + root, 0.0)
    o_ref[...] = jnp.dot(h2, wl_ref[...],
                         preferred_element_type=jnp.float32) + bl_ref[0]


def kernel(x, edge_index, W1_rel, b1, W1_root, W2_rel, b2, W2_root, Wl, bl):
    n, d_in = x.shape
    h1 = W1_rel.shape[1]
    h2 = W2_rel.shape[1]
    e = edge_index.shape[1]

    t_ch = -(-e // _CH)
    n0 = max(2, (int(t_ch * 0.655) // (_NS * 2)) * 2)
    n1 = max(2, (-(-(t_ch - _NS * n0) // (_NS * 2))) * 2)
    e0 = _NS * n0 * _CH
    e_pad = e0 + _NS * n1 * _CH
    n_pad = -(-(n + 1) // (_NS * _CH)) * (_NS * _CH)

    def _pack(a, fill):
        flat = jnp.concatenate(
            [a, jnp.full((e_pad - e,), fill, jnp.int32)])
        p0 = flat[:e0].reshape(_NS, n0, _CH)
        p1 = flat[e0:].reshape(_NS, n1, _CH)
        p1 = jnp.pad(p1, ((0, 0), (0, n0 - n1), (0, 0)))
        return jnp.concatenate([p0, p1], axis=0)

    src = _pack(edge_index[0], 0)
    dst = _pack(edge_index[1], n)

    xr = pl.pallas_call(
        _k1, out_shape=jax.ShapeDtypeStruct((n, h1 // 2), jnp.int32))(
            x, W1_rel)

    p1 = _make_seg_sum(n_pad, h1, n0, n1, True)(xr, src, dst)

    h, hr = pl.pallas_call(
        _k3,
        out_shape=[jax.ShapeDtypeStruct((n, h1), jnp.float32),
                   jax.ShapeDtypeStruct((n, h2 // 2), jnp.int32)],
    )(p1, x, W1_root, b1, W2_rel)

    p2 = _make_seg_sum(n_pad, h2, n0, n1, False)(hr, src, dst)

    out = pl.pallas_call(
        _k5,
        out_shape=jax.ShapeDtypeStruct((n, 1), jnp.float32),
    )(p2, p1, h, W2_root, b2, Wl, bl)
    return out


# speedup vs baseline: 1.1571x; 1.1331x over previous
import functools

import numpy as np

import jax
import jax.numpy as jnp
from jax import lax
from jax.experimental import pallas as pl
from jax.experimental.pallas import tpu as pltpu
from jax.experimental.pallas import tpu_sc as plsc

_NC = 2
_NS = 16
_NW = _NC * _NS
_CH = 128


def _make_seg_sum(n_pad, d, n0, n1, n1_last, with_counts):
    t0 = _NS * n0
    rpt = n_pad // _NS
    ncp = rpt // _CH
    dd = d + 16 if with_counts else d
    mesh = plsc.VectorSubcoreMesh(core_axis_name="c", subcore_axis_name="s")

    outs = jax.ShapeDtypeStruct((_NC, n_pad, dd), jnp.float32)
    scratch = [
        pltpu.VMEM((n0, _CH), jnp.int32),
        pltpu.VMEM((n0, _CH), jnp.int32),
        pltpu.VMEM((_CH, d // 2), jnp.int32),
        pltpu.VMEM((_CH, d // 2), jnp.int32),
        pltpu.VMEM((_CH, dd), jnp.float32),
        pltpu.VMEM((_CH, dd), jnp.float32),
        pltpu.VMEM_SHARED((n_pad, dd), jnp.float32),
        pltpu.SemaphoreType.DMA,
        pltpu.SemaphoreType.DMA,
        pltpu.SemaphoreType.DMA,
        pltpu.SemaphoreType.DMA,
    ]

    def load_idx(idx_hbm, c, s, src_v, dst_v):
        @pl.when(c == 0)
        def _():
            off = s * n0
            pltpu.sync_copy(idx_hbm.at[0, pl.ds(off, n0)], src_v)
            pltpu.sync_copy(idx_hbm.at[1, pl.ds(off, n0)], dst_v)

        @pl.when((c == 1) & (s < _NS - 1))
        def _():
            off = t0 + s * n1
            pltpu.sync_copy(idx_hbm.at[0, pl.ds(off, n1)],
                            src_v.at[pl.ds(0, n1)])
            pltpu.sync_copy(idx_hbm.at[1, pl.ds(off, n1)],
                            dst_v.at[pl.ds(0, n1)])

        @pl.when((c == 1) & (s == _NS - 1))
        def _():
            off = t0 + (_NS - 1) * n1
            pltpu.sync_copy(idx_hbm.at[0, pl.ds(off, n1_last)],
                            src_v.at[pl.ds(0, n1_last)])
            pltpu.sync_copy(idx_hbm.at[1, pl.ds(off, n1_last)],
                            dst_v.at[pl.ds(0, n1_last)])

    def pipeline(rows_hbm, src_v, dst_v, bfA, bfB, bufA, bufB, acc_sh,
                 gsA, gsB, ssA, ssB, nch, per_chunk):
        pltpu.async_copy(rows_hbm.at[src_v.at[0]], bfA, gsA)
        pltpu.async_copy(rows_hbm.at[src_v.at[1]], bfB, gsB)

        def convert(bf, buf):
            @plsc.parallel_loop(0, _CH, unroll=4)
            def crow(i):
                for g in range(d // 32):
                    vi = bf[i, pl.ds(16 * g, 16)]
                    lo = lax.bitcast_convert_type(
                        lax.shift_left(vi, 16), jnp.float32)
                    hi = lax.bitcast_convert_type(
                        lax.bitwise_and(vi, jnp.int32(-65536)), jnp.float32)
                    buf[i, pl.ds(16 * g, 16)] = lo
                    buf[i, pl.ds(d // 2 + 16 * g, 16)] = hi

        def half(j, bf, buf, gs, ss):
            pltpu.make_async_copy(rows_hbm.at[src_v.at[j]], bf, gs).wait()
            convert(bf, buf)

            @pl.when(j + 2 < nch)
            def _():
                pltpu.async_copy(rows_hbm.at[src_v.at[j + 2]], bf, gs)

            dsc = pltpu.async_copy(buf, acc_sh.at[dst_v.at[j]], ss, add=True)
            per_chunk(j)
            dsc.wait()

        def step(t, carry):
            half(2 * t, bfA, bufA, gsA, ssA)
            half(2 * t + 1, bfB, bufB, gsB, ssB)
            return carry
        lax.fori_loop(0, nch // 2, step, 0)

        @pl.when(nch % 2 == 1)
        def _():
            half(nch - 1, bfA, bufA, gsA, ssA)

    def body(rows_hbm, idx_hbm, out_hbm,
             src_v, dst_v, bfA, bfB, bufA, bufB, acc_sh,
             gsA, gsB, ssA, ssB):
        zero16 = jnp.zeros((16,), jnp.float32)
        one16 = jnp.ones((16,), jnp.float32)
        c = lax.axis_index("c")
        s = lax.axis_index("s")
        nch = jnp.where(c == 0, n0, jnp.where(s < _NS - 1, n1, n1_last))
        load_idx(idx_hbm, c, s, src_v, dst_v)

        def zrow(i, carry):
            for cc in range(dd // 16):
                bufA[i, pl.ds(cc * 16, 16)] = zero16
            return carry
        lax.fori_loop(0, _CH, zrow, 0)

        r0 = s * rpt
        for i in range(ncp):
            pltpu.sync_copy(bufA, acc_sh.at[pl.ds(r0 + i * _CH, _CH)])

        if with_counts:
            def orow(i, carry):
                bufA[i, pl.ds(d, 16)] = one16
                bufB[i, pl.ds(d, 16)] = one16
                return carry
            lax.fori_loop(0, _CH, orow, 0)
        plsc.subcore_barrier()

        pipeline(rows_hbm, src_v, dst_v, bfA, bfB, bufA, bufB, acc_sh,
                 gsA, gsB, ssA, ssB, nch, lambda j: None)
        plsc.subcore_barrier()

        for i in range(ncp):
            sl = pl.ds(r0 + i * _CH, _CH)
            pltpu.sync_copy(acc_sh.at[sl], bufA)
            pltpu.sync_copy(bufA, out_hbm.at[c, sl])

    return pl.kernel(
        body, mesh=mesh, out_type=outs, scratch_types=scratch,
        compiler_params=pltpu.CompilerParams(use_tc_tiling_on_sc=False))


def _pack_bf16_pairs(xf):
    k = xf.shape[1] // 2
    one = jnp.uint32(1)
    half = jnp.uint32(0x7FFF)
    sixteen = jnp.uint32(16)
    ul = lax.bitcast_convert_type(xf[:, :k], jnp.uint32)
    ur = lax.bitcast_convert_type(xf[:, k:], jnp.uint32)
    tl = lax.shift_right_logical(
        ul + half + (lax.shift_right_logical(ul, sixteen) & one), sixteen)
    tr = lax.shift_right_logical(
        ur + half + (lax.shift_right_logical(ur, sixteen) & one), sixteen)
    return lax.bitcast_convert_type(
        tl | lax.shift_left(tr, sixteen), jnp.int32)


def _k1(x_ref, w_ref, o_ref):
    o_ref[...] = _pack_bf16_pairs(
        jnp.dot(x_ref[...], w_ref[...], preferred_element_type=jnp.float32))


def _k3(p_ref, x_ref, w1root_ref, b1_ref, w2rel_ref, h_ref, hr_ref):
    n = x_ref.shape[0]
    k = w1root_ref.shape[1]
    agg = p_ref[0, :n, :k] + p_ref[1, :n, :k]
    cnt = p_ref[0, :n, k:k + 1] + p_ref[1, :n, k:k + 1]
    inv = 1.0 / jnp.maximum(cnt, 1.0)
    root = jnp.dot(x_ref[...], w1root_ref[...],
                   preferred_element_type=jnp.float32)
    h = jnp.maximum(agg * inv + b1_ref[...][None, :] + root, 0.0)
    h_ref[...] = h
    hr_ref[...] = _pack_bf16_pairs(
        jnp.dot(h, w2rel_ref[...], preferred_element_type=jnp.float32))


def _k5(p_ref, c_ref, h_ref, w2root_ref, b2_ref, wl_ref, bl_ref, o_ref):
    n = h_ref.shape[0]
    k = h_ref.shape[1]
    agg = p_ref[0, :n, :] + p_ref[1, :n, :]
    cnt = c_ref[0, :n, k:k + 1] + c_ref[1, :n, k:k + 1]
    inv = 1.0 / jnp.maximum(cnt, 1.0)
    root = jnp.dot(h_ref[...], w2root_ref[...],
                   preferred_element_type=jnp.float32)
    h2 = jnp.maximum(agg * inv + b2_ref[...][None, :] + root, 0.0)
    o_ref[...] = jnp.dot(h2, wl_ref[...],
                         preferred_element_type=jnp.float32) + bl_ref[0]


def kernel(x, edge_index, W1_rel, b1, W1_root, W2_rel, b2, W2_root, Wl, bl):
    n, d_in = x.shape
    h1 = W1_rel.shape[1]
    h2 = W2_rel.shape[1]
    e = edge_index.shape[1]

    if e % _CH:
        pad = -e % _CH
        edge_index = jnp.concatenate(
            [edge_index,
             jnp.stack([jnp.zeros((pad,), jnp.int32),
                        jnp.full((pad,), n, jnp.int32)])], axis=1)
    t_ch = edge_index.shape[1] // _CH
    idx3 = edge_index.reshape(2, t_ch, _CH)
    n0 = max(1, (int(t_ch * 0.655) // _NS))
    t1 = t_ch - _NS * n0
    n1 = -(-t1 // _NS)
    n1_last = t1 - (_NS - 1) * n1
    n_pad = -(-(n + 1) // (_NS * _CH)) * (_NS * _CH)

    xr = pl.pallas_call(
        _k1, out_shape=jax.ShapeDtypeStruct((n, h1 // 2), jnp.int32))(
            x, W1_rel)

    p1 = _make_seg_sum(n_pad, h1, n0, n1, n1_last, True)(xr, idx3)

    h, hr = pl.pallas_call(
        _k3,
        out_shape=[jax.ShapeDtypeStruct((n, h1), jnp.float32),
                   jax.ShapeDtypeStruct((n, h2 // 2), jnp.int32)],
    )(p1, x, W1_root, b1, W2_rel)

    p2 = _make_seg_sum(n_pad, h2, n0, n1, n1_last, False)(hr, idx3)

    out = pl.pallas_call(
        _k5,
        out_shape=jax.ShapeDtypeStruct((n, 1), jnp.float32),
    )(p2, p1, h, W2_root, b2, Wl, bl)
    return out


# speedup vs baseline: 1.2882x; 1.1133x over previous
import functools

import numpy as np

import jax
import jax.numpy as jnp
from jax import lax
from jax.experimental import pallas as pl
from jax.experimental.pallas import tpu as pltpu
from jax.experimental.pallas import tpu_sc as plsc

_NC = 2
_NS = 16
_NW = _NC * _NS
_CH = 128


def _make_seg_sum(n_pad, d, n0, n1, n1_last, with_counts):
    t0 = _NS * n0
    rpt = n_pad // _NS
    ncp = rpt // _CH
    dd = d + 16 if with_counts else d
    mesh = plsc.VectorSubcoreMesh(core_axis_name="c", subcore_axis_name="s")

    outs = jax.ShapeDtypeStruct((_NC, n_pad, dd), jnp.float32)
    scratch = [
        pltpu.VMEM((n0, _CH), jnp.int32),
        pltpu.VMEM((n0, _CH), jnp.int32),
        pltpu.VMEM((_CH, d // 2), jnp.int32),
        pltpu.VMEM((_CH, d // 2), jnp.int32),
        pltpu.VMEM((_CH, dd), jnp.float32),
        pltpu.VMEM((_CH, dd), jnp.float32),
        pltpu.VMEM_SHARED((n_pad, dd), jnp.float32),
        pltpu.SemaphoreType.DMA,
        pltpu.SemaphoreType.DMA,
        pltpu.SemaphoreType.DMA,
        pltpu.SemaphoreType.DMA,
    ]

    def load_idx(idx_hbm, c, s, src_v, dst_v):
        @pl.when(c == 0)
        def _():
            off = s * n0
            pltpu.sync_copy(idx_hbm.at[0, pl.ds(off, n0)], src_v)
            pltpu.sync_copy(idx_hbm.at[1, pl.ds(off, n0)], dst_v)

        @pl.when((c == 1) & (s < _NS - 1))
        def _():
            off = t0 + s * n1
            pltpu.sync_copy(idx_hbm.at[0, pl.ds(off, n1)],
                            src_v.at[pl.ds(0, n1)])
            pltpu.sync_copy(idx_hbm.at[1, pl.ds(off, n1)],
                            dst_v.at[pl.ds(0, n1)])

        @pl.when((c == 1) & (s == _NS - 1))
        def _():
            off = t0 + (_NS - 1) * n1
            pltpu.sync_copy(idx_hbm.at[0, pl.ds(off, n1_last)],
                            src_v.at[pl.ds(0, n1_last)])
            pltpu.sync_copy(idx_hbm.at[1, pl.ds(off, n1_last)],
                            dst_v.at[pl.ds(0, n1_last)])

    def pipeline(rows_hbm, src_v, dst_v, bfA, bfB, bufA, bufB, acc_sh,
                 gsA, gsB, ssA, ssB, nch, per_chunk):
        pltpu.async_copy(rows_hbm.at[src_v.at[0]], bfA, gsA)
        pltpu.async_copy(rows_hbm.at[src_v.at[1]], bfB, gsB)

        def convert(bf, buf):
            @plsc.parallel_loop(0, _CH, unroll=4)
            def crow(i):
                for g in range(d // 32):
                    vi = bf[i, pl.ds(16 * g, 16)]
                    lo = lax.bitcast_convert_type(
                        lax.shift_left(vi, 16), jnp.float32)
                    hi = lax.bitcast_convert_type(
                        lax.bitwise_and(vi, jnp.int32(-65536)), jnp.float32)
                    buf[i, pl.ds(16 * g, 16)] = lo
                    buf[i, pl.ds(d // 2 + 16 * g, 16)] = hi

        def half(j, bf, buf, gs, ss):
            pltpu.make_async_copy(rows_hbm.at[src_v.at[j]], bf, gs).wait()
            convert(bf, buf)

            @pl.when(j + 2 < nch)
            def _():
                pltpu.async_copy(rows_hbm.at[src_v.at[j + 2]], bf, gs)

            dsc = pltpu.async_copy(buf, acc_sh.at[dst_v.at[j]], ss, add=True)
            per_chunk(j)
            dsc.wait()

        def step(t, carry):
            half(2 * t, bfA, bufA, gsA, ssA)
            half(2 * t + 1, bfB, bufB, gsB, ssB)
            return carry
        lax.fori_loop(0, nch // 2, step, 0)

        @pl.when(nch % 2 == 1)
        def _():
            half(nch - 1, bfA, bufA, gsA, ssA)

    def body(rows_hbm, idx_hbm, out_hbm,
             src_v, dst_v, bfA, bfB, bufA, bufB, acc_sh,
             gsA, gsB, ssA, ssB):
        zero16 = jnp.zeros((16,), jnp.float32)
        one16 = jnp.ones((16,), jnp.float32)
        c = lax.axis_index("c")
        s = lax.axis_index("s")
        nch = jnp.where(c == 0, n0, jnp.where(s < _NS - 1, n1, n1_last))
        load_idx(idx_hbm, c, s, src_v, dst_v)

        def zrow(i, carry):
            for cc in range(dd // 16):
                bufA[i, pl.ds(cc * 16, 16)] = zero16
            return carry
        lax.fori_loop(0, _CH, zrow, 0)

        r0 = s * rpt
        for i in range(ncp):
            pltpu.sync_copy(bufA, acc_sh.at[pl.ds(r0 + i * _CH, _CH)])

        if with_counts:
            def orow(i, carry):
                bufA[i, pl.ds(d, 16)] = one16
                bufB[i, pl.ds(d, 16)] = one16
                return carry
            lax.fori_loop(0, _CH, orow, 0)
        plsc.subcore_barrier()

        pipeline(rows_hbm, src_v, dst_v, bfA, bfB, bufA, bufB, acc_sh,
                 gsA, gsB, ssA, ssB, nch, lambda j: None)
        plsc.subcore_barrier()

        for i in range(ncp):
            sl = pl.ds(r0 + i * _CH, _CH)
            pltpu.sync_copy(acc_sh.at[sl], bufA)
            pltpu.sync_copy(bufA, out_hbm.at[c, sl])

    return pl.kernel(
        body, mesh=mesh, out_type=outs, scratch_types=scratch,
        compiler_params=pltpu.CompilerParams(use_tc_tiling_on_sc=False))


def _pack_bf16_pairs(xf):
    k = xf.shape[1] // 2
    one = jnp.uint32(1)
    half = jnp.uint32(0x7FFF)
    sixteen = jnp.uint32(16)
    ul = lax.bitcast_convert_type(xf[:, :k], jnp.uint32)
    ur = lax.bitcast_convert_type(xf[:, k:], jnp.uint32)
    tl = lax.shift_right_logical(
        ul + half + (lax.shift_right_logical(ul, sixteen) & one), sixteen)
    tr = lax.shift_right_logical(
        ur + half + (lax.shift_right_logical(ur, sixteen) & one), sixteen)
    return lax.bitcast_convert_type(
        tl | lax.shift_left(tr, sixteen), jnp.int32)


def _k1(x_ref, w_ref, o_ref):
    o_ref[...] = _pack_bf16_pairs(
        jnp.dot(x_ref[...], w_ref[...], preferred_element_type=jnp.float32))


def _k3(p_ref, x_ref, w1root_ref, b1_ref, w2rel_ref, h_ref, hr_ref):
    n = x_ref.shape[0]
    k = w1root_ref.shape[1]
    agg = p_ref[0, :n, :k] + p_ref[1, :n, :k]
    cnt = p_ref[0, :n, k:k + 1] + p_ref[1, :n, k:k + 1]
    inv = 1.0 / jnp.maximum(cnt, 1.0)
    root = jnp.dot(x_ref[...], w1root_ref[...],
                   preferred_element_type=jnp.float32)
    h = jnp.maximum(agg * inv + b1_ref[...][None, :] + root, 0.0)
    h_ref[...] = h
    hr_ref[...] = _pack_bf16_pairs(
        jnp.dot(h, w2rel_ref[...], preferred_element_type=jnp.float32))


def _k5(p_ref, c_ref, h_ref, w2root_ref, b2_ref, wl_ref, bl_ref, o_ref):
    n = h_ref.shape[0]
    k = h_ref.shape[1]
    agg = p_ref[0, :n, :] + p_ref[1, :n, :]
    cnt = c_ref[0, :n, k:k + 1] + c_ref[1, :n, k:k + 1]
    inv = 1.0 / jnp.maximum(cnt, 1.0)
    root = jnp.dot(h_ref[...], w2root_ref[...],
                   preferred_element_type=jnp.float32)
    h2 = jnp.maximum(agg * inv + b2_ref[...][None, :] + root, 0.0)
    o_ref[...] = jnp.dot(h2, wl_ref[...],
                         preferred_element_type=jnp.float32) + bl_ref[0]


def kernel(x, edge_index, W1_rel, b1, W1_root, W2_rel, b2, W2_root, Wl, bl):
    n, d_in = x.shape
    h1 = W1_rel.shape[1]
    h2 = W2_rel.shape[1]
    e = edge_index.shape[1]

    if e % _CH:
        pad = -e % _CH
        edge_index = jnp.concatenate(
            [edge_index,
             jnp.stack([jnp.zeros((pad,), jnp.int32),
                        jnp.full((pad,), n, jnp.int32)])], axis=1)
    t_ch = edge_index.shape[1] // _CH
    idx3 = edge_index.reshape(2, t_ch, _CH)
    n0 = max(1, (int(t_ch * 0.527) // _NS))
    t1 = t_ch - _NS * n0
    n1 = -(-t1 // _NS)
    n1_last = t1 - (_NS - 1) * n1
    n_pad = -(-(n + 1) // (_NS * _CH)) * (_NS * _CH)

    xr = pl.pallas_call(
        _k1, out_shape=jax.ShapeDtypeStruct((n, h1 // 2), jnp.int32))(
            x, W1_rel)

    p1 = _make_seg_sum(n_pad, h1, n0, n1, n1_last, True)(xr, idx3)

    h, hr = pl.pallas_call(
        _k3,
        out_shape=[jax.ShapeDtypeStruct((n, h1), jnp.float32),
                   jax.ShapeDtypeStruct((n, h2 // 2), jnp.int32)],
    )(p1, x, W1_root, b1, W2_rel)

    p2 = _make_seg_sum(n_pad, h2, n0, n1, n1_last, False)(hr, idx3)

    out = pl.pallas_call(
        _k5,
        out_shape=jax.ShapeDtypeStruct((n, 1), jnp.float32),
    )(p2, p1, h, W2_root, b2, Wl, bl)
    return out


# speedup vs baseline: 1.2883x; 1.0000x over previous
import jax
import jax.numpy as jnp
from jax import lax
from jax.experimental import pallas as pl
from jax.experimental.pallas import tpu as pltpu
from jax.experimental.pallas import tpu_sc as plsc

_NC = 2
_NS = 16
_NW = _NC * _NS
_CH = 128


def _make_seg_sum(n_pad, d, n0, n1, n1_last, with_counts):
    t0 = _NS * n0
    rpt = n_pad // _NS
    ncp = rpt // _CH
    dd = d + 16 if with_counts else d
    mesh = plsc.VectorSubcoreMesh(core_axis_name="c", subcore_axis_name="s")

    outs = jax.ShapeDtypeStruct((_NC, n_pad, dd), jnp.float32)
    scratch = [
        pltpu.VMEM((n0, _CH), jnp.int32),
        pltpu.VMEM((n0, _CH), jnp.int32),
        pltpu.VMEM((_CH, d // 2), jnp.int32),
        pltpu.VMEM((_CH, d // 2), jnp.int32),
        pltpu.VMEM((_CH, dd), jnp.float32),
        pltpu.VMEM((_CH, dd), jnp.float32),
        pltpu.VMEM_SHARED((n_pad, dd), jnp.float32),
        pltpu.SemaphoreType.DMA,
        pltpu.SemaphoreType.DMA,
        pltpu.SemaphoreType.DMA,
        pltpu.SemaphoreType.DMA,
    ]

    def load_idx(idx_hbm, c, s, src_v, dst_v):
        @pl.when(c == 0)
        def _():
            off = s * n0
            pltpu.sync_copy(idx_hbm.at[0, pl.ds(off, n0)], src_v)
            pltpu.sync_copy(idx_hbm.at[1, pl.ds(off, n0)], dst_v)

        @pl.when((c == 1) & (s < _NS - 1))
        def _():
            off = t0 + s * n1
            pltpu.sync_copy(idx_hbm.at[0, pl.ds(off, n1)],
                            src_v.at[pl.ds(0, n1)])
            pltpu.sync_copy(idx_hbm.at[1, pl.ds(off, n1)],
                            dst_v.at[pl.ds(0, n1)])

        @pl.when((c == 1) & (s == _NS - 1))
        def _():
            off = t0 + (_NS - 1) * n1
            pltpu.sync_copy(idx_hbm.at[0, pl.ds(off, n1_last)],
                            src_v.at[pl.ds(0, n1_last)])
            pltpu.sync_copy(idx_hbm.at[1, pl.ds(off, n1_last)],
                            dst_v.at[pl.ds(0, n1_last)])

    def pipeline(rows_hbm, src_v, dst_v, bfA, bfB, bufA, bufB, acc_sh,
                 gsA, gsB, ssA, ssB, nch, per_chunk):
        pltpu.async_copy(rows_hbm.at[src_v.at[0]], bfA, gsA)
        pltpu.async_copy(rows_hbm.at[src_v.at[1]], bfB, gsB)

        def convert(bf, buf):
            @plsc.parallel_loop(0, _CH, unroll=4)
            def crow(i):
                for g in range(d // 32):
                    vi = bf[i, pl.ds(16 * g, 16)]
                    lo = lax.bitcast_convert_type(
                        lax.shift_left(vi, 16), jnp.float32)
                    hi = lax.bitcast_convert_type(
                        lax.bitwise_and(vi, jnp.int32(-65536)), jnp.float32)
                    buf[i, pl.ds(16 * g, 16)] = lo
                    buf[i, pl.ds(d // 2 + 16 * g, 16)] = hi

        def half(j, bf, buf, gs, ss):
            pltpu.make_async_copy(rows_hbm.at[src_v.at[j]], bf, gs).wait()
            convert(bf, buf)

            @pl.when(j + 2 < nch)
            def _():
                pltpu.async_copy(rows_hbm.at[src_v.at[j + 2]], bf, gs)

            dsc = pltpu.async_copy(buf, acc_sh.at[dst_v.at[j]], ss, add=True)
            per_chunk(j)
            dsc.wait()

        def step(t, carry):
            half(2 * t, bfA, bufA, gsA, ssA)
            half(2 * t + 1, bfB, bufB, gsB, ssB)
            return carry
        lax.fori_loop(0, nch // 2, step, 0)

        @pl.when(nch % 2 == 1)
        def _():
            half(nch - 1, bfA, bufA, gsA, ssA)

    def body(rows_hbm, idx_hbm, out_hbm,
             src_v, dst_v, bfA, bfB, bufA, bufB, acc_sh,
             gsA, gsB, ssA, ssB):
        zero16 = jnp.zeros((16,), jnp.float32)
        one16 = jnp.ones((16,), jnp.float32)
        c = lax.axis_index("c")
        s = lax.axis_index("s")
        nch = jnp.where(c == 0, n0, jnp.where(s < _NS - 1, n1, n1_last))
        load_idx(idx_hbm, c, s, src_v, dst_v)

        def zrow(i, carry):
            for cc in range(dd // 16):
                bufA[i, pl.ds(cc * 16, 16)] = zero16
            return carry
        lax.fori_loop(0, _CH, zrow, 0)

        r0 = s * rpt
        for i in range(ncp):
            pltpu.sync_copy(bufA, acc_sh.at[pl.ds(r0 + i * _CH, _CH)])

        if with_counts:
            def orow(i, carry):
                bufA[i, pl.ds(d, 16)] = one16
                bufB[i, pl.ds(d, 16)] = one16
                return carry
            lax.fori_loop(0, _CH, orow, 0)
        plsc.subcore_barrier()

        pipeline(rows_hbm, src_v, dst_v, bfA, bfB, bufA, bufB, acc_sh,
                 gsA, gsB, ssA, ssB, nch, lambda j: None)
        plsc.subcore_barrier()

        for i in range(ncp):
            sl = pl.ds(r0 + i * _CH, _CH)
            pltpu.sync_copy(acc_sh.at[sl], bufA)
            pltpu.sync_copy(bufA, out_hbm.at[c, sl])

    return pl.kernel(
        body, mesh=mesh, out_type=outs, scratch_types=scratch,
        compiler_params=pltpu.CompilerParams(use_tc_tiling_on_sc=False))


def _pack_bf16_pairs(xf):
    k = xf.shape[1] // 2
    one = jnp.uint32(1)
    half = jnp.uint32(0x7FFF)
    sixteen = jnp.uint32(16)
    ul = lax.bitcast_convert_type(xf[:, :k], jnp.uint32)
    ur = lax.bitcast_convert_type(xf[:, k:], jnp.uint32)
    tl = lax.shift_right_logical(
        ul + half + (lax.shift_right_logical(ul, sixteen) & one), sixteen)
    tr = lax.shift_right_logical(
        ur + half + (lax.shift_right_logical(ur, sixteen) & one), sixteen)
    return lax.bitcast_convert_type(
        tl | lax.shift_left(tr, sixteen), jnp.int32)


def _k1(x_ref, w_ref, o_ref):
    o_ref[...] = _pack_bf16_pairs(
        jnp.dot(x_ref[...], w_ref[...], preferred_element_type=jnp.float32))


def _k3(p_ref, x_ref, w1root_ref, b1_ref, w2rel_ref, h_ref, hr_ref):
    n = x_ref.shape[0]
    k = w1root_ref.shape[1]
    agg = p_ref[0, :n, :k] + p_ref[1, :n, :k]
    cnt = p_ref[0, :n, k:k + 1] + p_ref[1, :n, k:k + 1]
    inv = 1.0 / jnp.maximum(cnt, 1.0)
    root = jnp.dot(x_ref[...], w1root_ref[...],
                   preferred_element_type=jnp.float32)
    h = jnp.maximum(agg * inv + b1_ref[...][None, :] + root, 0.0)
    h_ref[...] = h
    hr_ref[...] = _pack_bf16_pairs(
        jnp.dot(h, w2rel_ref[...], preferred_element_type=jnp.float32))


def _k5(p_ref, c_ref, h_ref, w2root_ref, b2_ref, wl_ref, bl_ref, o_ref):
    n = h_ref.shape[0]
    k = h_ref.shape[1]
    agg = p_ref[0, :n, :] + p_ref[1, :n, :]
    cnt = c_ref[0, :n, k:k + 1] + c_ref[1, :n, k:k + 1]
    inv = 1.0 / jnp.maximum(cnt, 1.0)
    root = jnp.dot(h_ref[...], w2root_ref[...],
                   preferred_element_type=jnp.float32)
    h2 = jnp.maximum(agg * inv + b2_ref[...][None, :] + root, 0.0)
    o_ref[...] = jnp.dot(h2, wl_ref[...],
                         preferred_element_type=jnp.float32) + bl_ref[0]


def kernel(x, edge_index, W1_rel, b1, W1_root, W2_rel, b2, W2_root, Wl, bl):
    n, d_in = x.shape
    h1 = W1_rel.shape[1]
    h2 = W2_rel.shape[1]
    e = edge_index.shape[1]

    if e % _CH:
        pad = -e % _CH
        edge_index = jnp.concatenate(
            [edge_index,
             jnp.stack([jnp.zeros((pad,), jnp.int32),
                        jnp.full((pad,), n, jnp.int32)])], axis=1)
    t_ch = edge_index.shape[1] // _CH
    idx3 = edge_index.reshape(2, t_ch, _CH)
    n0 = max(1, (int(t_ch * 0.527) // _NS))
    t1 = t_ch - _NS * n0
    n1 = -(-t1 // _NS)
    n1_last = t1 - (_NS - 1) * n1
    n_pad = -(-(n + 1) // (_NS * _CH)) * (_NS * _CH)

    xr = pl.pallas_call(
        _k1, out_shape=jax.ShapeDtypeStruct((n, h1 // 2), jnp.int32))(
            x, W1_rel)

    p1 = _make_seg_sum(n_pad, h1, n0, n1, n1_last, True)(xr, idx3)

    h, hr = pl.pallas_call(
        _k3,
        out_shape=[jax.ShapeDtypeStruct((n, h1), jnp.float32),
                   jax.ShapeDtypeStruct((n, h2 // 2), jnp.int32)],
    )(p1, x, W1_root, b1, W2_rel)

    p2 = _make_seg_sum(n_pad, h2, n0, n1, n1_last, False)(hr, idx3)

    out = pl.pallas_call(
        _k5,
        out_shape=jax.ShapeDtypeStruct((n, 1), jnp.float32),
    )(p2, p1, h, W2_root, b2, Wl, bl)
    return out
